# baseline (speedup 1.0000x reference)
import functools

import jax
import jax.numpy as jnp
from jax import lax
from jax.experimental import pallas as pl
from jax.experimental.pallas import tpu as pltpu
from jax.experimental.pallas import tpu_sc as plsc

Q = 4096
K = 16384
D = 256
P = 16384
KNN_K = 10

_QB = 2048
_KB = 4096
_PB = 512
_W = 128

_BIG = 3e38


def _stage1_body(f_ref, m_ref, o_ref):
    j = pl.program_id(1)
    f = f_ref[...]
    m = m_ref[...]
    ones = jnp.ones((1, D), jnp.float32)
    m2 = jax.lax.dot_general(
        ones, m * m, (((1,), (1,)), ((), ())),
        preferred_element_type=jnp.float32)
    prod = jax.lax.dot_general(
        f, m, (((1,), (1,)), ((), ())),
        preferred_element_type=jnp.float32)
    t = m2 - 2.0 * prod
    acc = t[:, :_W]
    for k in range(1, _KB // _W):
        acc = jnp.minimum(acc, t[:, k * _W:(k + 1) * _W])
    rmin = jnp.min(acc, axis=1, keepdims=True)
    prev = jnp.where(j == 0, _BIG, o_ref[...])
    accmin = jnp.minimum(prev, rmin)
    f2 = jnp.sum(f * f, axis=1, keepdims=True)
    o_ref[...] = jnp.where(j == (K // _KB) - 1,
                           jnp.sqrt(jnp.maximum(accmin + f2, 0.0)), accmin)


def _stage2_body(pa_ref, ca_ref, rhs_ref, fs_ref, mx_ref):
    b = pl.program_id(0)
    pa = pa_ref[...]
    ca = ca_ref[...]
    dc = jax.lax.dot_general(
        pa, ca, (((1,), (1,)), ((), ())),
        preferred_element_type=jnp.float32)
    nchunk = Q // _W
    big = jnp.full((_PB, _W), _BIG, jnp.float32)
    a0, a1, a2, a3 = big, big, big, big
    for k in range(nchunk):
        x = dc[:, k * _W:(k + 1) * _W]
        h0 = jnp.maximum(a0, x)
        a0 = jnp.minimum(a0, x)
        h1 = jnp.maximum(a1, h0)
        a1 = jnp.minimum(a1, h0)
        h2 = jnp.maximum(a2, h1)
        a2 = jnp.minimum(a2, h1)
        a3 = jnp.minimum(a3, h2)
    t = jnp.full((_PB, 1), -_BIG, jnp.float32)
    for _ in range(KNN_K):
        head = jnp.where(a0 > t, a0,
               jnp.where(a1 > t, a1,
               jnp.where(a2 > t, a2,
               jnp.where(a3 > t, a3, _BIG))))
        t = jnp.min(head, axis=1, keepdims=True)
    bad = jnp.any(a3 < t)

    def _direct(_):
        td = jnp.full((_PB, 1), -_BIG, jnp.float32)
        for _ in range(KNN_K):
            acc = jnp.full((_PB, _W), _BIG, jnp.float32)
            for k in range(nchunk):
                c = dc[:, k * _W:(k + 1) * _W]
                acc = jnp.minimum(acc, jnp.where(c > td, c, _BIG))
            td = jnp.min(acc, axis=1, keepdims=True)
        return td

    t10 = jax.lax.cond(bad, _direct, lambda _: t, None)
    rhs = rhs_ref[...]
    lt01 = jnp.where(dc < t10, 1.0, 0.0)
    eq01 = jnp.where(dc == t10, 1.0, 0.0)
    r_lt = jax.lax.dot_general(
        lt01, rhs, (((1,), (0,)), ((), ())),
        preferred_element_type=jnp.float32)
    r_eq = jax.lax.dot_general(
        eq01, rhs, (((1,), (0,)), ((), ())),
        preferred_element_type=jnp.float32)
    cnt_lt = r_lt[:, 0:1]
    sum_lt = r_lt[:, 1:2]
    cnt_eq = jnp.maximum(r_eq[:, 0:1], 1.0)
    sum_eq = r_eq[:, 1:2]
    full = (sum_lt + (KNN_K - cnt_lt) * sum_eq / cnt_eq) * (1.0 / KNN_K)
    fs_ref[...] = full
    blockmax = jnp.max(full, axis=0, keepdims=True)
    prevmx = jnp.where(b == 0, jnp.full((1, 1), -_BIG, jnp.float32),
                       mx_ref[...])
    mx_ref[...] = jnp.maximum(prevmx, blockmax)


_NC, _NS, _L = 2, 16, 16
_NW = _NC * _NS
_SPLIT = 11776
_SC_ROWS = P - _SPLIT
_RPT = _SC_ROWS // _NW
_RI = 16
_NCHUNK = Q // _L


def _sc_body(px_h, py_h, pz_h, cx_h, cy_h, cz_h, c2_h, s_h, out_h,
             cx_v, cy_v, cz_v, c2_v, s_v, px_v, py_v, pz_v, out_v):
    wid = lax.axis_index("s") * _NC + lax.axis_index("c")
    base = pl.multiple_of(wid * _RPT, _RPT)
    pltpu.sync_copy(cx_h, cx_v)
    pltpu.sync_copy(cy_h, cy_v)
    pltpu.sync_copy(cz_h, cz_v)
    pltpu.sync_copy(c2_h, c2_v)
    pltpu.sync_copy(s_h, s_v)
    pltpu.sync_copy(px_h.at[pl.ds(_SPLIT + base, _RPT)], px_v)
    pltpu.sync_copy(py_h.at[pl.ds(_SPLIT + base, _RPT)], py_v)
    pltpu.sync_copy(pz_h.at[pl.ds(_SPLIT + base, _RPT)], pz_v)
    lane = lax.iota(jnp.int32, _L)
    mask10 = lane >= (_L - KNN_K)

    def rowblk_body(rb, _):
        r0 = rb * _RI
        axs, ays, azs = [], [], []
        for i in range(_RI):
            ridx = jnp.full((_L,), r0 + i, jnp.int32)
            axs.append(plsc.load_gather(px_v, [ridx]) * -2.0)
            ays.append(plsc.load_gather(py_v, [ridx]) * -2.0)
            azs.append(plsc.load_gather(pz_v, [ridx]) * -2.0)
        bk0 = tuple(jnp.full((_L,), _BIG, jnp.float32) for _ in range(_RI))
        bi0 = tuple(jnp.zeros((_L,), jnp.int32) for _ in range(_RI))

        def chunk_body(j, carry):
            bks, bis = carry
            o = pl.ds(j * _L, _L)
            cxv = cx_v[o]
            cyv = cy_v[o]
            czv = cz_v[o]
            c2v = c2_v[o]
            idxv = j * _L + lane
            nbk, nbi = [], []
            for i in range(_RI):
                d = c2v + cxv * axs[i] + cyv * ays[i] + czv * azs[i]
                dk, di = plsc.sort_key_val(d, idxv)
                m = dk < bks[i]
                lo = jnp.where(m, dk, bks[i])
                loi = jnp.where(m, di, bis[i])
                k2, i2 = plsc.sort_key_val(lo, loi, descending=True)
                nbk.append(k2)
                nbi.append(i2)
            return tuple(nbk), tuple(nbi)

        _, bis = lax.fori_loop(0, _NCHUNK, chunk_body, (bk0, bi0))
        res = jnp.zeros((_L,), jnp.float32)
        for i in range(_RI):
            sv = plsc.load_gather(s_v, [bis[i]])
            tot = jnp.sum(jnp.where(mask10, sv, 0.0)) * (1.0 / KNN_K)
            res = jnp.where(lane == i, tot, res)
        out_v[pl.ds(r0, _L)] = res
        return 0

    lax.fori_loop(0, _RPT // _RI, rowblk_body, 0)
    pltpu.sync_copy(out_v, out_h.at[pl.ds(_SPLIT + base, _RPT)])


_sc_knn = pl.kernel(
    _sc_body,
    out_type=jax.ShapeDtypeStruct((P,), jnp.float32),
    mesh=plsc.VectorSubcoreMesh(core_axis_name="c", subcore_axis_name="s"),
    compiler_params=pltpu.CompilerParams(needs_layout_passes=False),
    scratch_types=[
        pltpu.VMEM((Q,), jnp.float32),
        pltpu.VMEM((Q,), jnp.float32),
        pltpu.VMEM((Q,), jnp.float32),
        pltpu.VMEM((Q,), jnp.float32),
        pltpu.VMEM((Q,), jnp.float32),
        pltpu.VMEM((_RPT,), jnp.float32),
        pltpu.VMEM((_RPT,), jnp.float32),
        pltpu.VMEM((_RPT,), jnp.float32),
        pltpu.VMEM((_RPT,), jnp.float32),
    ],
)


def _max_body(x_ref, o_ref):
    o_ref[...] = jnp.max(jnp.max(x_ref[...], axis=1, keepdims=True),
                         axis=0, keepdims=True)


@functools.partial(jax.jit)
def kernel(features, memory_features, centers, points):
    center_scores = pl.pallas_call(
        _stage1_body,
        grid=(Q // _QB, K // _KB),
        in_specs=[
            pl.BlockSpec((_QB, D), lambda i, j: (i, 0)),
            pl.BlockSpec((_KB, D), lambda i, j: (j, 0)),
        ],
        out_specs=pl.BlockSpec((_QB, 1), lambda i, j: (i, 0)),
        out_shape=jax.ShapeDtypeStruct((Q, 1), jnp.float32),
    )(features, memory_features)

    c2 = jnp.sum(centers * centers, axis=1, keepdims=True)
    sc_out = _sc_knn(points[:, 0], points[:, 1], points[:, 2],
                     centers[:, 0], centers[:, 1], centers[:, 2],
                     c2.reshape(Q), center_scores.reshape(Q))
    sc_scores = sc_out[_SPLIT:]

    zeros_p = jnp.zeros((_SPLIT, 4), jnp.float32)
    pa = jnp.concatenate(
        [-2.0 * points[:_SPLIT], jnp.ones((_SPLIT, 1), jnp.float32),
         zeros_p], axis=1)
    ca = jnp.concatenate(
        [centers, c2, jnp.zeros((Q, 4), jnp.float32)], axis=1)
    rhs = jnp.concatenate(
        [jnp.ones((Q, 1), jnp.float32), center_scores], axis=1)
    tc2d, _ = pl.pallas_call(
        _stage2_body,
        grid=(_SPLIT // _PB,),
        in_specs=[
            pl.BlockSpec((_PB, 8), lambda b: (b, 0)),
            pl.BlockSpec((Q, 8), lambda b: (0, 0)),
            pl.BlockSpec((Q, 2), lambda b: (0, 0)),
        ],
        out_specs=[
            pl.BlockSpec((_PB, 1), lambda b: (b, 0)),
            pl.BlockSpec((1, 1), lambda b: (0, 0)),
        ],
        out_shape=[
            jax.ShapeDtypeStruct((_SPLIT, 1), jnp.float32),
            jax.ShapeDtypeStruct((1, 1), jnp.float32),
        ],
    )(pa, ca, rhs)

    full_scores = jnp.concatenate([tc2d.reshape(_SPLIT), sc_scores])

    mx = pl.pallas_call(
        _max_body,
        out_shape=jax.ShapeDtypeStruct((1, 1), jnp.float32),
    )(full_scores.reshape(128, 128))

    return full_scores, mx.reshape(())


# speedup vs baseline: 1.1371x; 1.1371x over previous
import functools

import jax
import jax.numpy as jnp
from jax import lax
from jax.experimental import pallas as pl
from jax.experimental.pallas import tpu as pltpu
from jax.experimental.pallas import tpu_sc as plsc

Q = 4096
K = 16384
D = 256
P = 16384
KNN_K = 10

_QB = 2048
_KB = 4096
_PB = 512
_W = 128

_BIG = 3e38


def _stage1_body(f_ref, m_ref, o_ref):
    j = pl.program_id(1)
    f = f_ref[...]
    m = m_ref[...]
    ones = jnp.ones((1, D), jnp.float32)
    m2 = jax.lax.dot_general(
        ones, m * m, (((1,), (1,)), ((), ())),
        preferred_element_type=jnp.float32)
    prod = jax.lax.dot_general(
        f, m, (((1,), (1,)), ((), ())),
        preferred_element_type=jnp.float32)
    t = m2 - 2.0 * prod
    acc = t[:, :_W]
    for k in range(1, _KB // _W):
        acc = jnp.minimum(acc, t[:, k * _W:(k + 1) * _W])
    rmin = jnp.min(acc, axis=1, keepdims=True)
    prev = jnp.where(j == 0, _BIG, o_ref[...])
    accmin = jnp.minimum(prev, rmin)
    f2 = jnp.sum(f * f, axis=1, keepdims=True)
    o_ref[...] = jnp.where(j == (K // _KB) - 1,
                           jnp.sqrt(jnp.maximum(accmin + f2, 0.0)), accmin)


def _stage2_body(pa_ref, ca_ref, rhs_ref, fs_ref, mx_ref):
    b = pl.program_id(0)
    pa = pa_ref[...]
    ca = ca_ref[...]
    dc = jax.lax.dot_general(
        pa, ca, (((1,), (1,)), ((), ())),
        preferred_element_type=jnp.float32)
    nchunk = Q // _W
    big = jnp.full((_PB, _W), _BIG, jnp.float32)
    a0, a1, a2, a3 = big, big, big, big
    for k in range(nchunk):
        x = dc[:, k * _W:(k + 1) * _W]
        h0 = jnp.maximum(a0, x)
        a0 = jnp.minimum(a0, x)
        h1 = jnp.maximum(a1, h0)
        a1 = jnp.minimum(a1, h0)
        h2 = jnp.maximum(a2, h1)
        a2 = jnp.minimum(a2, h1)
        a3 = jnp.minimum(a3, h2)
    t = jnp.full((_PB, 1), -_BIG, jnp.float32)
    for _ in range(KNN_K):
        head = jnp.where(a0 > t, a0,
               jnp.where(a1 > t, a1,
               jnp.where(a2 > t, a2,
               jnp.where(a3 > t, a3, _BIG))))
        t = jnp.min(head, axis=1, keepdims=True)
    bad = jnp.any(a3 < t)

    def _direct(_):
        td = jnp.full((_PB, 1), -_BIG, jnp.float32)
        for _ in range(KNN_K):
            acc = jnp.full((_PB, _W), _BIG, jnp.float32)
            for k in range(nchunk):
                c = dc[:, k * _W:(k + 1) * _W]
                acc = jnp.minimum(acc, jnp.where(c > td, c, _BIG))
            td = jnp.min(acc, axis=1, keepdims=True)
        return td

    t10 = jax.lax.cond(bad, _direct, lambda _: t, None)
    rhs = rhs_ref[...]
    lt01 = jnp.where(dc < t10, 1.0, 0.0)
    eq01 = jnp.where(dc == t10, 1.0, 0.0)
    r_lt = jax.lax.dot_general(
        lt01, rhs, (((1,), (0,)), ((), ())),
        preferred_element_type=jnp.float32)
    r_eq = jax.lax.dot_general(
        eq01, rhs, (((1,), (0,)), ((), ())),
        preferred_element_type=jnp.float32)
    cnt_lt = r_lt[:, 0:1]
    sum_lt = r_lt[:, 1:2]
    cnt_eq = jnp.maximum(r_eq[:, 0:1], 1.0)
    sum_eq = r_eq[:, 1:2]
    full = (sum_lt + (KNN_K - cnt_lt) * sum_eq / cnt_eq) * (1.0 / KNN_K)
    fs_ref[...] = full
    blockmax = jnp.max(full, axis=0, keepdims=True)
    prevmx = jnp.where(b == 0, jnp.full((1, 1), -_BIG, jnp.float32),
                       mx_ref[...])
    mx_ref[...] = jnp.maximum(prevmx, blockmax)


_NC, _NS, _L = 2, 16, 16
_NW = _NC * _NS
_SPLIT = 12800
_SC_ROWS = P - _SPLIT
_RPT = _SC_ROWS // _NW
_RI = 16
_NCHUNK = Q // _L


def _sc_body(px_h, py_h, pz_h, cx_h, cy_h, cz_h, c2_h, s_h, out_h,
             cx_v, cy_v, cz_v, c2_v, s_v, px_v, py_v, pz_v, out_v):
    wid = lax.axis_index("s") * _NC + lax.axis_index("c")
    base = pl.multiple_of(wid * _RPT, _RPT)
    pltpu.sync_copy(cx_h, cx_v)
    pltpu.sync_copy(cy_h, cy_v)
    pltpu.sync_copy(cz_h, cz_v)
    pltpu.sync_copy(c2_h, c2_v)
    pltpu.sync_copy(s_h, s_v)
    pltpu.sync_copy(px_h.at[pl.ds(_SPLIT + base, _RPT)], px_v)
    pltpu.sync_copy(py_h.at[pl.ds(_SPLIT + base, _RPT)], py_v)
    pltpu.sync_copy(pz_h.at[pl.ds(_SPLIT + base, _RPT)], pz_v)
    lane = lax.iota(jnp.int32, _L)
    mask10 = lane >= (_L - KNN_K)

    def rowblk_body(rb, _):
        r0 = rb * _RI
        axs, ays, azs = [], [], []
        for i in range(_RI):
            ridx = jnp.full((_L,), r0 + i, jnp.int32)
            axs.append(plsc.load_gather(px_v, [ridx]) * -2.0)
            ays.append(plsc.load_gather(py_v, [ridx]) * -2.0)
            azs.append(plsc.load_gather(pz_v, [ridx]) * -2.0)
        bk0 = tuple(jnp.full((_L,), _BIG, jnp.float32) for _ in range(_RI))
        bi0 = tuple(jnp.zeros((_L,), jnp.int32) for _ in range(_RI))

        def chunk_body(j, carry):
            bks, bis = carry
            o = pl.ds(j * _L, _L)
            cxv = cx_v[o]
            cyv = cy_v[o]
            czv = cz_v[o]
            c2v = c2_v[o]
            idxv = j * _L + lane
            nbk, nbi = [], []
            for i in range(_RI):
                d = c2v + cxv * axs[i] + cyv * ays[i] + czv * azs[i]
                dk, di = plsc.sort_key_val(d, idxv)
                m = dk < bks[i]
                lo = jnp.where(m, dk, bks[i])
                loi = jnp.where(m, di, bis[i])
                k2, i2 = plsc.sort_key_val(lo, loi, descending=True)
                nbk.append(k2)
                nbi.append(i2)
            return tuple(nbk), tuple(nbi)

        _, bis = lax.fori_loop(0, _NCHUNK, chunk_body, (bk0, bi0))
        res = jnp.zeros((_L,), jnp.float32)
        for i in range(_RI):
            sv = plsc.load_gather(s_v, [bis[i]])
            tot = jnp.sum(jnp.where(mask10, sv, 0.0)) * (1.0 / KNN_K)
            res = jnp.where(lane == i, tot, res)
        out_v[pl.ds(r0, _L)] = res
        return 0

    lax.fori_loop(0, _RPT // _RI, rowblk_body, 0)
    pltpu.sync_copy(out_v, out_h.at[pl.ds(_SPLIT + base, _RPT)])


_sc_knn = pl.kernel(
    _sc_body,
    out_type=jax.ShapeDtypeStruct((P,), jnp.float32),
    mesh=plsc.VectorSubcoreMesh(core_axis_name="c", subcore_axis_name="s"),
    compiler_params=pltpu.CompilerParams(needs_layout_passes=False),
    scratch_types=[
        pltpu.VMEM((Q,), jnp.float32),
        pltpu.VMEM((Q,), jnp.float32),
        pltpu.VMEM((Q,), jnp.float32),
        pltpu.VMEM((Q,), jnp.float32),
        pltpu.VMEM((Q,), jnp.float32),
        pltpu.VMEM((_RPT,), jnp.float32),
        pltpu.VMEM((_RPT,), jnp.float32),
        pltpu.VMEM((_RPT,), jnp.float32),
        pltpu.VMEM((_RPT,), jnp.float32),
    ],
)


def _max_body(x_ref, o_ref):
    o_ref[...] = jnp.max(jnp.max(x_ref[...], axis=1, keepdims=True),
                         axis=0, keepdims=True)


@functools.partial(jax.jit)
def kernel(features, memory_features, centers, points):
    center_scores = pl.pallas_call(
        _stage1_body,
        grid=(Q // _QB, K // _KB),
        in_specs=[
            pl.BlockSpec((_QB, D), lambda i, j: (i, 0)),
            pl.BlockSpec((_KB, D), lambda i, j: (j, 0)),
        ],
        out_specs=pl.BlockSpec((_QB, 1), lambda i, j: (i, 0)),
        out_shape=jax.ShapeDtypeStruct((Q, 1), jnp.float32),
    )(features, memory_features)

    c2 = jnp.sum(centers * centers, axis=1, keepdims=True)
    sc_out = _sc_knn(points[:, 0], points[:, 1], points[:, 2],
                     centers[:, 0], centers[:, 1], centers[:, 2],
                     c2.reshape(Q), center_scores.reshape(Q))
    sc_scores = sc_out[_SPLIT:]

    zeros_p = jnp.zeros((_SPLIT, 4), jnp.float32)
    pa = jnp.concatenate(
        [-2.0 * points[:_SPLIT], jnp.ones((_SPLIT, 1), jnp.float32),
         zeros_p], axis=1)
    ca = jnp.concatenate(
        [centers, c2, jnp.zeros((Q, 4), jnp.float32)], axis=1)
    rhs = jnp.concatenate(
        [jnp.ones((Q, 1), jnp.float32), center_scores], axis=1)
    tc2d, _ = pl.pallas_call(
        _stage2_body,
        grid=(_SPLIT // _PB,),
        in_specs=[
            pl.BlockSpec((_PB, 8), lambda b: (b, 0)),
            pl.BlockSpec((Q, 8), lambda b: (0, 0)),
            pl.BlockSpec((Q, 2), lambda b: (0, 0)),
        ],
        out_specs=[
            pl.BlockSpec((_PB, 1), lambda b: (b, 0)),
            pl.BlockSpec((1, 1), lambda b: (0, 0)),
        ],
        out_shape=[
            jax.ShapeDtypeStruct((_SPLIT, 1), jnp.float32),
            jax.ShapeDtypeStruct((1, 1), jnp.float32),
        ],
    )(pa, ca, rhs)

    full_scores = jnp.concatenate([tc2d.reshape(_SPLIT), sc_scores])

    mx = pl.pallas_call(
        _max_body,
        out_shape=jax.ShapeDtypeStruct((1, 1), jnp.float32),
    )(full_scores.reshape(128, 128))

    return full_scores, mx.reshape(())
